# baseline (speedup 1.0000x reference)
import jax
import jax.numpy as jnp
from jax import lax
from jax.experimental import pallas as pl
from jax.experimental.pallas import tpu as pltpu
from jax.experimental.pallas import tpu_sc as plsc

N = 2_000_000
B = 500_000
RT = 8.31446261815324 * 298.15

NC, NS, L = 2, 16, 16
ROWS = N // 128
G = 32
FULL_I = 15
VPB = G * 128 // L

I_CID, I_CCNT, I_SEEN, I_FID = 0, 1, 2, 3


def _sc_body(ids_hbm, lg_hbm, out_hbm, idx_v, lg_v, frec_v, t2_v, ssq_v,
             obuf, smi, smf):
    c = lax.axis_index("c")
    s = lax.axis_index("s")
    iota = lax.broadcasted_iota(jnp.int32, (L,), 0)
    zero = jnp.zeros((L,), jnp.float32)
    sh_r = jnp.maximum(iota - 1, 0)
    sh_l = jnp.minimum(iota + 1, L - 1)

    t2_v[...] = zero
    ssq_v[...] = zero
    frec_v[...] = zero
    smi[I_CID] = -1
    smi[I_CCNT] = 0
    smi[I_SEEN] = 0
    smf[0] = 0.0

    base = jnp.where(c == 0, 0, 7813)
    rem = jnp.where(c == 0, 5, 4)
    start = base + 488 * s + jnp.minimum(s, rem)
    tail9 = s < rem

    def _do_vec(j):
        d = idx_v[pl.ds(j * L, L)]
        lgv = lg_v[pl.ds(j * L, L)]
        g = lgv * RT
        cot = jnp.ones((L,), jnp.float32)
        vj = cot * g - g
        ssq_v[...] = ssq_v[...] + vj * vj

        cid = smi[I_CID]
        ccnt = smi[I_CCNT]
        seen = smi[I_SEEN]
        csum = smf[0]

        zi = jnp.zeros((L,), jnp.int32)
        d0 = jnp.sum(jnp.where(iota == 0, d, zi))
        closed = (d0 != cid) & (ccnt > 0)
        ccnt_f = ccnt.astype(jnp.float32)
        one = jnp.ones((L,), jnp.float32)
        csum_v = jnp.where(iota == 0, csum, zero)
        den_v = jnp.maximum(jnp.where(iota == 0, ccnt_f, one), one)
        carry_v = csum_v * csum_v / den_v
        t2_v[...] = t2_v[...] + jnp.where(closed & (seen != 0), carry_v, zero)

        @pl.when(closed & (seen == 0))
        def _rec_carry():
            frec_v[...] = (jnp.where(iota == 0, csum, zero)
                           + jnp.where(iota == 1, ccnt_f, zero))

        seen = seen | closed.astype(jnp.int32)

        d_prev = jnp.where(iota == 0, cid, jnp.take(d, sh_r))
        m_start = d != d_prev
        sidx = plsc.cummax(jnp.where(m_start, iota, jnp.full((L,), -1,
                                                             jnp.int32)))
        cs = plsc.cumsum(vj)
        base_g = jnp.take(cs, jnp.maximum(sidx - 1, 0))
        run_base = jnp.where(sidx > 0, base_g,
                             jnp.where(sidx == 0, zero, zero - csum))
        seg = cs - run_base
        cnt_i = jnp.where(sidx >= 0, iota - sidx + 1, iota + 1 + ccnt)
        cnt_f = cnt_i.astype(jnp.float32)

        nstart = jnp.take(m_start.astype(jnp.int32), sh_l)
        m_end = (nstart == 1) & (iota != L - 1)
        raw = seg * seg / cnt_f
        e0 = jnp.min(jnp.where(m_end, iota, jnp.full((L,), L, jnp.int32)))
        first_now = (seen == 0) & (e0 < L)
        keep = m_end & jnp.logical_not((iota == e0) & first_now)
        t2_v[...] = t2_v[...] + jnp.where(keep, raw, zero)

        @pl.when(first_now)
        def _rec():
            fs = jnp.sum(jnp.where(iota == e0, seg, zero))
            fc = jnp.sum(jnp.where(iota == e0, cnt_f, zero))
            frec_v[...] = (jnp.where(iota == 0, fs, zero)
                           + jnp.where(iota == 1, fc, zero))

        smi[I_SEEN] = seen | (e0 < L).astype(jnp.int32)
        last = iota == L - 1
        smi[I_CID] = jnp.sum(jnp.where(last, d, jnp.zeros((L,), jnp.int32)))
        smi[I_CCNT] = jnp.sum(jnp.where(last, cnt_i,
                                        jnp.zeros((L,), jnp.int32)))
        smf[0] = jnp.sum(jnp.where(last, seg, zero))

    def _vec_loop(j, carry):
        _do_vec(j)
        return carry

    for i in range(FULL_I):
        row = start + G * i
        pltpu.sync_copy(ids_hbm.at[pl.ds(row * 128, G * 128)], idx_v)
        pltpu.sync_copy(lg_hbm.at[pl.ds(row * 128, G * 128)], lg_v)
        if i == 0:
            d0 = idx_v[pl.ds(0, L)]
            smi[I_FID] = jnp.sum(jnp.where(iota == 0, d0,
                                           jnp.zeros((L,), jnp.int32)))
        lax.fori_loop(0, VPB, _vec_loop, 0)
    trow = start + G * FULL_I
    pltpu.sync_copy(ids_hbm.at[pl.ds(trow * 128, 8 * 128)],
                    idx_v.at[pl.ds(0, 8 * 128)])
    pltpu.sync_copy(lg_hbm.at[pl.ds(trow * 128, 8 * 128)],
                    lg_v.at[pl.ds(0, 8 * 128)])
    lax.fori_loop(0, 8 * 128 // L, _vec_loop, 0)

    @pl.when(tail9)
    def _tail():
        pltpu.sync_copy(ids_hbm.at[pl.ds((trow + 8) * 128, 128)],
                        idx_v.at[pl.ds(0, 128)])
        pltpu.sync_copy(lg_hbm.at[pl.ds((trow + 8) * 128, 128)],
                        lg_v.at[pl.ds(0, 128)])
        lax.fori_loop(0, 128 // L, _vec_loop, 0)

    frec = frec_v[...]
    fs = jnp.sum(jnp.where(iota == 0, frec, zero))
    fc = jnp.sum(jnp.where(iota == 1, frec, zero))
    vals = [jnp.sum(t2_v[...]),
            jnp.sum(ssq_v[...]),
            smi[I_SEEN].astype(jnp.float32),
            smi[I_FID].astype(jnp.float32),
            fs,
            fc,
            smi[I_CID].astype(jnp.float32),
            smf[0],
            smi[I_CCNT].astype(jnp.float32)]
    res = zero
    for k, v in enumerate(vals):
        res = res + jnp.where(iota == k, v, zero)
    obuf[...] = res
    wid = c * NS + s
    pltpu.sync_copy(obuf, out_hbm.at[wid])


@jax.jit
def _gd_loss_sc(ids1, lg1):
    mesh = plsc.VectorSubcoreMesh(core_axis_name="c", subcore_axis_name="s")
    f = pl.kernel(
        _sc_body,
        out_type=jax.ShapeDtypeStruct((NC * NS, L), jnp.float32),
        mesh=mesh,
        compiler_params=pltpu.CompilerParams(use_tc_tiling_on_sc=False,
                                             needs_layout_passes=False),
        scratch_types=[
            pltpu.VMEM((G * 128,), jnp.int32),
            pltpu.VMEM((G * 128,), jnp.float32),
            pltpu.VMEM((L,), jnp.float32),
            pltpu.VMEM((L,), jnp.float32),
            pltpu.VMEM((L,), jnp.float32),
            pltpu.VMEM((L,), jnp.float32),
            pltpu.SMEM((4,), jnp.int32),
            pltpu.SMEM((1,), jnp.float32),
        ],
    )
    rows = f(ids1, lg1)
    t2 = jnp.sum(rows[:, 0])
    ssq = jnp.sum(rows[:, 1])
    pval = jnp.concatenate([rows[:, 2] > 0.5,
                            jnp.ones((NC * NS,), jnp.bool_)])
    pid = jnp.concatenate([rows[:, 3], rows[:, 6]])
    psum = jnp.where(pval, jnp.concatenate([rows[:, 4], rows[:, 7]]), 0.0)
    pcnt = jnp.where(pval, jnp.concatenate([rows[:, 5], rows[:, 8]]), 0.0)
    pid = jnp.where(pval, pid, -1.0)
    eq = (pid[:, None] == pid[None, :]) & pval[:, None] & pval[None, :]
    eqf = eq.astype(jnp.float32)
    S = jnp.sum(eqf * psum[None, :], axis=1)
    C = jnp.sum(eqf * pcnt[None, :], axis=1)
    k = jnp.arange(2 * NC * NS)
    earlier = eq & (k[None, :] < k[:, None])
    first = pval & jnp.logical_not(jnp.any(earlier, axis=1))
    t2 = t2 + jnp.sum(jnp.where(first, S * S / jnp.maximum(C, 1.0), 0.0))
    return (ssq - t2) / jnp.float32(B)


def kernel(component_mole_frac, component_batch_batch, ln_gamma_calc):
    del component_mole_frac
    lg1 = ln_gamma_calc.reshape(N)
    return _gd_loss_sc(component_batch_batch, lg1)
